# baseline (speedup 1.0000x reference)
import functools

import jax
import jax.numpy as jnp
from jax import lax
from jax.experimental import pallas as pl
from jax.experimental.pallas import tpu as pltpu
from jax.experimental.pallas import tpu_sc as plsc

_TOTAL_HEADS = 32
_NC, _NS = 2, 16


def _tc_body(licp_ref, xk_ref, xv_ref, kc_ref, vl_ref,
             ko_ref, keys_ref, vals_ref):
    bs = ko_ref.shape[2]
    insert = xk_ref.shape[1]
    heads = ko_ref.shape[3]
    rep = _TOTAL_HEADS // heads
    li = licp_ref[0]
    cp = licp_ref[1]
    start = pl.program_id(1) * bs

    ko_ref[...] = kc_ref[...]
    for i in range(insert):
        lr = cp + i - start
        @pl.when((lr >= 0) & (lr < bs))
        def _():
            ko_ref[li, 0, lr] = xk_ref[0, i]

    kl = ko_ref[li, 0]
    vl = vl_ref[0, 0]
    for i in range(insert):
        lr = cp + i - start
        @pl.when((lr >= 0) & (lr < bs))
        def _():
            vl_ref[0, 0, lr] = xv_ref[0, i]
    vl = vl_ref[0, 0]
    for h in range(heads):
        keys_ref[0, :, h * rep:(h + 1) * rep, :] = jnp.broadcast_to(
            kl[:, h:h + 1, :], (bs, rep, kl.shape[2]))
        vals_ref[0, :, h * rep:(h + 1) * rep, :] = jnp.broadcast_to(
            vl[:, h:h + 1, :], (bs, rep, vl.shape[2]))


def _tc_part(xk, xv, k_cache, v_cache, licp2):
    L, B, S, H, D = k_cache.shape
    insert = xk.shape[1]
    bs = 512
    grid = (B, S // bs)
    kc_spec = pl.BlockSpec((L, 1, bs, H, D), lambda b, s, ref: (0, b, s, 0, 0))
    vl_spec = pl.BlockSpec((1, 1, bs, H, D),
                           lambda b, s, ref: (ref[0], b, s, 0, 0))
    x_spec = pl.BlockSpec((1, insert, H, D), lambda b, s, ref: (b, 0, 0, 0))
    out_spec = pl.BlockSpec((1, bs, _TOTAL_HEADS, D),
                            lambda b, s, ref: (b, s, 0, 0))
    grid_spec = pltpu.PrefetchScalarGridSpec(
        num_scalar_prefetch=1,
        grid=grid,
        in_specs=[x_spec, x_spec, kc_spec, vl_spec],
        out_specs=[kc_spec, out_spec, out_spec],
    )
    return pl.pallas_call(
        _tc_body,
        grid_spec=grid_spec,
        out_shape=[
            jax.ShapeDtypeStruct(k_cache.shape, k_cache.dtype),
            jax.ShapeDtypeStruct((B, S, _TOTAL_HEADS, D), xk.dtype),
            jax.ShapeDtypeStruct((B, S, _TOTAL_HEADS, D), xv.dtype),
        ],
        compiler_params=pltpu.CompilerParams(
            dimension_semantics=("parallel", "parallel"),
        ),
    )(licp2, xk, xv, k_cache, v_cache)


def _sc_vcache_copy(xv, v_cache, licp16):
    L, B, S, H, D = v_cache.shape
    insert = xv.shape[1]
    ch = 128

    mesh = plsc.VectorSubcoreMesh(core_axis_name="c", subcore_axis_name="s")

    @functools.partial(
        pl.kernel,
        out_type=jax.ShapeDtypeStruct(v_cache.shape, v_cache.dtype),
        mesh=mesh,
        scratch_types=[
            pltpu.VMEM((ch, H, D), v_cache.dtype),
            pltpu.VMEM((insert, H, D), xv.dtype),
            pltpu.VMEM((16,), jnp.int32),
            pltpu.SemaphoreType.DMA,
            pltpu.SemaphoreType.DMA,
        ],
    )
    def sc_kernel(licp_hbm, xv_hbm, vc_hbm, vo_hbm,
                  chunk_v, xbuf_v, licp_v, gsem, ssem):
        wid = lax.axis_index("s") * _NC + lax.axis_index("c")
        l = wid // B
        b = wid % B
        pltpu.sync_copy(licp_hbm, licp_v)
        licp_vec = licp_v[...]
        li = licp_vec[0]
        cp = licp_vec[1]

        pltpu.async_copy(vc_hbm.at[l, b], vo_hbm.at[l, b], gsem).wait()

        @pl.when(l == li)
        def _():
            pltpu.async_copy(xv_hbm.at[b], vo_hbm.at[l, b, pl.ds(cp, insert)],
                             ssem).wait()

    return sc_kernel(licp16, xv, v_cache)


def kernel(xk, xv, k_cache, v_cache, layer_idx, cur_pos, n_rep):
    L, B, S, H, D = k_cache.shape
    insert = xk.shape[1]
    li = jnp.clip(jnp.asarray(layer_idx, jnp.int32), 0, L - 1)
    cp = jnp.clip(jnp.asarray(cur_pos, jnp.int32), 0, S - insert)
    licp2 = jnp.stack([li, cp])
    licp16 = jnp.zeros((16,), jnp.int32).at[0].set(li).at[1].set(cp)
    vo = _sc_vcache_copy(xv, v_cache, licp16)
    ko, keys, values = _tc_part(xk, xv, k_cache, v_cache, licp2)
    return keys, values, ko, vo


# speedup vs baseline: 16.0710x; 16.0710x over previous
import jax
import jax.numpy as jnp
from jax.experimental import pallas as pl
from jax.experimental.pallas import tpu as pltpu

_TOTAL_HEADS = 32


def _body(li_ref, cp_ref, xk_ref, xv_ref, kc_ref, vc_ref,
          ko_ref, vo_ref, keys_ref, vals_ref):
    bs = ko_ref.shape[2]
    insert = xk_ref.shape[1]
    heads = ko_ref.shape[3]
    rep = _TOTAL_HEADS // heads
    li = li_ref[0]
    cp = cp_ref[0]
    start = pl.program_id(0) * bs

    ko_ref[...] = kc_ref[...]
    vo_ref[...] = vc_ref[...]

    for i in range(insert):
        lr = cp + i - start
        @pl.when((lr >= 0) & (lr < bs))
        def _():
            ko_ref[li, 0, lr] = xk_ref[0, i]
            vo_ref[li, 0, lr] = xv_ref[0, i]

    kl = ko_ref[li, 0]
    vl = vo_ref[li, 0]
    for h in range(heads):
        keys_ref[0, :, h * rep:(h + 1) * rep, :] = jnp.broadcast_to(
            kl[:, h:h + 1, :], (bs, rep, kl.shape[2]))
        vals_ref[0, :, h * rep:(h + 1) * rep, :] = jnp.broadcast_to(
            vl[:, h:h + 1, :], (bs, rep, vl.shape[2]))


def kernel(xk, xv, k_cache, v_cache, layer_idx, cur_pos, n_rep):
    L, B, S, H, D = k_cache.shape
    insert = xk.shape[1]
    bs = 512
    li = jnp.clip(jnp.asarray(layer_idx, jnp.int32), 0, L - 1).reshape(1)
    cp = jnp.clip(jnp.asarray(cur_pos, jnp.int32), 0, S - insert).reshape(1)

    grid = (S // bs, B)
    cache_spec = pl.BlockSpec((L, 1, bs, H, D), lambda s, b: (0, b, s, 0, 0))
    x_spec = pl.BlockSpec((1, insert, H, D), lambda s, b: (b, 0, 0, 0))
    out_spec = pl.BlockSpec((1, bs, _TOTAL_HEADS, D), lambda s, b: (b, s, 0, 0))

    ko, vo, keys, values = pl.pallas_call(
        _body,
        grid=grid,
        in_specs=[
            pl.BlockSpec(memory_space=pltpu.SMEM),
            pl.BlockSpec(memory_space=pltpu.SMEM),
            x_spec, x_spec, cache_spec, cache_spec,
        ],
        out_specs=[
            cache_spec, cache_spec, out_spec, out_spec,
        ],
        out_shape=[
            jax.ShapeDtypeStruct(k_cache.shape, k_cache.dtype),
            jax.ShapeDtypeStruct(v_cache.shape, v_cache.dtype),
            jax.ShapeDtypeStruct((B, S, _TOTAL_HEADS, D), xk.dtype),
            jax.ShapeDtypeStruct((B, S, _TOTAL_HEADS, D), xv.dtype),
        ],
        compiler_params=pltpu.CompilerParams(
            dimension_semantics=("parallel", "parallel"),
        ),
    )(li, cp, xk, xv, k_cache, v_cache)
    return keys, values, ko, vo


# speedup vs baseline: 16.1654x; 1.0059x over previous
import jax
import jax.numpy as jnp
from jax.experimental import pallas as pl
from jax.experimental.pallas import tpu as pltpu

_TOTAL_HEADS = 32


def _body(li_ref, cp_ref, xk_ref, xv_ref, kc_ref, vc_ref,
          ko_ref, vo_ref, keys_ref, vals_ref):
    bs = ko_ref.shape[2]
    insert = xk_ref.shape[1]
    heads = ko_ref.shape[3]
    rep = _TOTAL_HEADS // heads
    li = li_ref[0]
    cp = cp_ref[0]
    start = pl.program_id(1) * bs

    ko_ref[...] = kc_ref[...]
    vo_ref[...] = vc_ref[...]

    for i in range(insert):
        lr = cp + i - start
        @pl.when((lr >= 0) & (lr < bs))
        def _():
            ko_ref[li, 0, lr] = xk_ref[0, i]
            vo_ref[li, 0, lr] = xv_ref[0, i]

    kl = ko_ref[li, 0]
    vl = vo_ref[li, 0]
    for h in range(heads):
        keys_ref[0, :, h * rep:(h + 1) * rep, :] = jnp.broadcast_to(
            kl[:, h:h + 1, :], (bs, rep, kl.shape[2]))
        vals_ref[0, :, h * rep:(h + 1) * rep, :] = jnp.broadcast_to(
            vl[:, h:h + 1, :], (bs, rep, vl.shape[2]))


def kernel(xk, xv, k_cache, v_cache, layer_idx, cur_pos, n_rep):
    L, B, S, H, D = k_cache.shape
    insert = xk.shape[1]
    bs = 512
    li = jnp.clip(jnp.asarray(layer_idx, jnp.int32), 0, L - 1).reshape(1)
    cp = jnp.clip(jnp.asarray(cur_pos, jnp.int32), 0, S - insert).reshape(1)

    grid = (B, S // bs)
    cache_spec = pl.BlockSpec((L, 1, bs, H, D), lambda b, s: (0, b, s, 0, 0))
    x_spec = pl.BlockSpec((1, insert, H, D), lambda b, s: (b, 0, 0, 0))
    out_spec = pl.BlockSpec((1, bs, _TOTAL_HEADS, D), lambda b, s: (b, s, 0, 0))

    ko, vo, keys, values = pl.pallas_call(
        _body,
        grid=grid,
        in_specs=[
            pl.BlockSpec(memory_space=pltpu.SMEM),
            pl.BlockSpec(memory_space=pltpu.SMEM),
            x_spec, x_spec, cache_spec, cache_spec,
        ],
        out_specs=[
            cache_spec, cache_spec, out_spec, out_spec,
        ],
        out_shape=[
            jax.ShapeDtypeStruct(k_cache.shape, k_cache.dtype),
            jax.ShapeDtypeStruct(v_cache.shape, v_cache.dtype),
            jax.ShapeDtypeStruct((B, S, _TOTAL_HEADS, D), xk.dtype),
            jax.ShapeDtypeStruct((B, S, _TOTAL_HEADS, D), xv.dtype),
        ],
        compiler_params=pltpu.CompilerParams(
            dimension_semantics=("parallel", "parallel"),
        ),
    )(li, cp, xk, xv, k_cache, v_cache)
    return keys, values, ko, vo
